# Initial kernel scaffold; baseline (speedup 1.0000x reference)
import jax
import jax.numpy as jnp
from jax.experimental import pallas as pl


def kernel(input_ids, token_type_ids, word_embeddings, position_embeddings, token_type_embeddings, gamma, beta):
    raise NotImplementedError("write your pallas kernel here")


# speedup vs baseline: 3.3621x; 3.3621x over previous
import functools

import jax
import jax.numpy as jnp
from jax import lax
from jax.experimental import pallas as pl
from jax.experimental.pallas import tpu as pltpu
from jax.experimental.pallas import tpu_sc as plsc

B, L, D, V = 1024, 200, 128, 100000
NC, NS = 2, 16
NW = NC * NS
N = B * L
TPW = N // NW
C = 256
NCHUNK = TPW // C
NR = D // 16
EPS = 1e-12


def _lane_sum(v):
    iota = lax.broadcasted_iota(jnp.int32, (16,), 0)
    for sh in (8, 4, 2, 1):
        v = v + v.at[iota ^ sh].get(mode="promise_in_bounds")
    return v


def _rsqrt_newton(x):
    i = lax.bitcast_convert_type(x, jnp.int32)
    y = lax.bitcast_convert_type(jnp.int32(0x5F3759DF) - (i >> 1), jnp.float32)
    for _ in range(3):
        y = y * (1.5 - 0.5 * x * y * y)
    return y


def _sc_kernel(ids_hbm, tts_hbm, wemb_hbm, pos_hbm, typ_hbm, gam_hbm, bet_hbm,
               out_hbm, idx_v, ttx_v, pt_v, x_v, typ_v, gb_v):
    wid = lax.axis_index("s") * NC + lax.axis_index("c")
    wbase = wid * TPW

    pltpu.sync_copy(pos_hbm.at[pl.ds(0, L)], pt_v)
    pltpu.sync_copy(typ_hbm, typ_v)
    pltpu.sync_copy(gam_hbm, gb_v.at[0])
    pltpu.sync_copy(bet_hbm, gb_v.at[1])

    t0 = [typ_v[0, pl.ds(rr * 16, 16)] for rr in range(NR)]
    t1 = [typ_v[1, pl.ds(rr * 16, 16)] for rr in range(NR)]
    dT = [t1[rr] - t0[rr] for rr in range(NR)]
    gv = [gb_v[0, pl.ds(rr * 16, 16)] for rr in range(NR)]
    bv = [gb_v[1, pl.ds(rr * 16, 16)] for rr in range(NR)]

    @pl.loop(0, L)
    def _(i):
        for rr in range(NR):
            s = pl.ds(rr * 16, 16)
            pt_v[i, s] = pt_v[i, s] + t0[rr]

    @pl.loop(0, NCHUNK)
    def _(c):
        gbase = wbase + c * C
        pltpu.sync_copy(ids_hbm.at[pl.ds(gbase, C)], idx_v)
        pltpu.sync_copy(tts_hbm.at[pl.ds(gbase, C)], ttx_v)
        pltpu.sync_copy(wemb_hbm.at[idx_v], x_v)

        @pl.loop(0, C)
        def _(i):
            pos = lax.rem(gbase + i, L)
            g = (i // 16) * 16
            lane = i - g
            ttg = ttx_v[pl.ds(pl.multiple_of(g, 16), 16)]
            tt = ttg.at[jnp.full((16,), lane, jnp.int32)].get(
                mode="promise_in_bounds")
            ttf = tt.astype(jnp.float32)

            xs = []
            s = None
            s2 = None
            for rr in range(NR):
                sl = pl.ds(rr * 16, 16)
                x = (x_v[i, sl] + pt_v[pos, sl]) + ttf * dT[rr]
                xs.append(x)
                s = x if s is None else s + x
                s2 = x * x if s2 is None else s2 + x * x

            s = _lane_sum(s)
            s2 = _lane_sum(s2)
            mu = s * (1.0 / D)
            var = s2 * (1.0 / D) - mu * mu
            rstd = _rsqrt_newton(var + EPS)

            for rr in range(NR):
                y = ((xs[rr] - mu) * rstd) * gv[rr] + bv[rr]
                x_v[i, pl.ds(rr * 16, 16)] = y

        pltpu.sync_copy(x_v, out_hbm.at[pl.ds(gbase, C)])


def kernel(input_ids, token_type_ids, word_embeddings, position_embeddings,
           token_type_embeddings, gamma, beta):
    mesh = plsc.VectorSubcoreMesh(core_axis_name="c", subcore_axis_name="s")
    run = functools.partial(
        pl.kernel,
        out_type=jax.ShapeDtypeStruct((N, D), jnp.float32),
        mesh=mesh,
        scratch_types=[
            pltpu.VMEM((C,), jnp.int32),
            pltpu.VMEM((C,), jnp.int32),
            pltpu.VMEM((L, D), jnp.float32),
            pltpu.VMEM((C, D), jnp.float32),
            pltpu.VMEM((2, D), jnp.float32),
            pltpu.VMEM((2, D), jnp.float32),
        ],
    )(_sc_kernel)
    out = run(input_ids.reshape(N).astype(jnp.int32),
              token_type_ids.reshape(N).astype(jnp.int32),
              word_embeddings, position_embeddings, token_type_embeddings,
              gamma, beta)
    return out.reshape(B, L, D)


# speedup vs baseline: 3.3904x; 1.0084x over previous
import functools

import jax
import jax.numpy as jnp
from jax import lax
from jax.experimental import pallas as pl
from jax.experimental.pallas import tpu as pltpu
from jax.experimental.pallas import tpu_sc as plsc

B, L, D, V = 1024, 200, 128, 100000
NC, NS = 2, 16
NW = NC * NS
N = B * L
TPW = N // NW
C = 128
NCHUNK = TPW // C
NPAIR = NCHUNK // 2
NR = D // 16
EPS = 1e-12


def _lane_sum(v):
    iota = lax.broadcasted_iota(jnp.int32, (16,), 0)
    for sh in (8, 4, 2, 1):
        v = v + v.at[iota ^ sh].get(mode="promise_in_bounds")
    return v


def _rsqrt_newton(x):
    i = lax.bitcast_convert_type(x, jnp.int32)
    y = lax.bitcast_convert_type(jnp.int32(0x5F3759DF) - (i >> 1), jnp.float32)
    for _ in range(2):
        y = y * (1.5 - 0.5 * x * y * y)
    return y


def _sc_kernel(ids_hbm, tts_hbm, wemb_hbm, pos_hbm, typ_hbm, gam_hbm, bet_hbm,
               out_hbm,
               idx0, idx1, ttx0, ttx1, x0, x1, y0, y1, pt_v, typ_v, gb_v,
               si0, si1, sg0, sg1, so0, so1):
    idx = (idx0, idx1)
    ttx = (ttx0, ttx1)
    xv = (x0, x1)
    yv = (y0, y1)
    si = (si0, si1)
    sg = (sg0, sg1)
    so = (so0, so1)

    wid = lax.axis_index("s") * NC + lax.axis_index("c")
    wbase = wid * TPW

    pltpu.sync_copy(pos_hbm.at[pl.ds(0, L)], pt_v)
    pltpu.sync_copy(typ_hbm, typ_v)
    pltpu.sync_copy(gam_hbm, gb_v.at[0])
    pltpu.sync_copy(bet_hbm, gb_v.at[1])

    t0 = [typ_v[0, pl.ds(rr * 16, 16)] for rr in range(NR)]
    t1 = [typ_v[1, pl.ds(rr * 16, 16)] for rr in range(NR)]
    dT = [t1[rr] - t0[rr] for rr in range(NR)]
    gv = [gb_v[0, pl.ds(rr * 16, 16)] for rr in range(NR)]
    bv = [gb_v[1, pl.ds(rr * 16, 16)] for rr in range(NR)]

    @pl.loop(0, L)
    def _(i):
        for rr in range(NR):
            s = pl.ds(rr * 16, 16)
            pt_v[i, s] = pt_v[i, s] + t0[rr]

    def launch_inputs(p, c):
        gb = wbase + c * C
        pltpu.async_copy(ids_hbm.at[pl.ds(gb, C)], idx[p], si[p])
        pltpu.async_copy(tts_hbm.at[pl.ds(gb, C)], ttx[p], si[p])

    def wait_inputs(p):
        pltpu.make_async_copy(ids_hbm.at[pl.ds(0, C)], idx[p], si[p]).wait()
        pltpu.make_async_copy(tts_hbm.at[pl.ds(0, C)], ttx[p], si[p]).wait()

    def launch_gather(p):
        pltpu.async_copy(wemb_hbm.at[idx[p]], xv[p], sg[p])

    def wait_gather(p):
        pltpu.make_async_copy(wemb_hbm.at[pl.ds(0, C)], xv[p], sg[p]).wait()

    def launch_out(p, c):
        gb = wbase + c * C
        pltpu.async_copy(yv[p], out_hbm.at[pl.ds(gb, C)], so[p])

    def wait_out(p):
        pltpu.make_async_copy(yv[p], out_hbm.at[pl.ds(0, C)], so[p]).wait()

    def compute(p, c):
        gb = wbase + c * C

        @pl.loop(0, C, unroll=2)
        def _(i):
            pos = lax.rem(gb + i, L)
            g = (i // 16) * 16
            lane = i - g
            ttg = ttx[p][pl.ds(pl.multiple_of(g, 16), 16)]
            tt = ttg.at[jnp.full((16,), lane, jnp.int32)].get(
                mode="promise_in_bounds")
            ttf = tt.astype(jnp.float32)

            xs = []
            s = None
            s2 = None
            for rr in range(NR):
                sl = pl.ds(rr * 16, 16)
                x = (xv[p][i, sl] + pt_v[pos, sl]) + ttf * dT[rr]
                xs.append(x)
                s = x if s is None else s + x
                s2 = x * x if s2 is None else s2 + x * x

            s = _lane_sum(s)
            s2 = _lane_sum(s2)
            mu = s * (1.0 / D)
            var = s2 * (1.0 / D) - mu * mu
            rstd = _rsqrt_newton(var + EPS)

            for rr in range(NR):
                y = ((xs[rr] - mu) * rstd) * gv[rr] + bv[rr]
                yv[p][i, pl.ds(rr * 16, 16)] = y

    def phase(p, c, first=False, no_next=False, no_next2=False):
        if not no_next:
            wait_inputs(1 - p)
            launch_gather(1 - p)
        wait_gather(p)
        if not (no_next or no_next2):
            launch_inputs(p, c + 2)
        if not first:
            wait_out(p)
        compute(p, c)
        launch_out(p, c)

    def phase_sync(p, c):
        launch_inputs(p, c)
        wait_inputs(p)
        launch_gather(p)
        wait_gather(p)
        compute(p, c)
        launch_out(p, c)
        wait_out(p)

    @pl.loop(0, NPAIR)
    def _(k):
        phase_sync(0, 2 * k)
        phase_sync(1, 2 * k + 1)


def kernel(input_ids, token_type_ids, word_embeddings, position_embeddings,
           token_type_embeddings, gamma, beta):
    mesh = plsc.VectorSubcoreMesh(core_axis_name="c", subcore_axis_name="s")
    run = functools.partial(
        pl.kernel,
        out_type=jax.ShapeDtypeStruct((N, D), jnp.float32),
        mesh=mesh,
        scratch_types=[
            pltpu.VMEM((C,), jnp.int32),
            pltpu.VMEM((C,), jnp.int32),
            pltpu.VMEM((C,), jnp.int32),
            pltpu.VMEM((C,), jnp.int32),
            pltpu.VMEM((C, D), jnp.float32),
            pltpu.VMEM((C, D), jnp.float32),
            pltpu.VMEM((C, D), jnp.float32),
            pltpu.VMEM((C, D), jnp.float32),
            pltpu.VMEM((L, D), jnp.float32),
            pltpu.VMEM((2, D), jnp.float32),
            pltpu.VMEM((2, D), jnp.float32),
            pltpu.SemaphoreType.DMA,
            pltpu.SemaphoreType.DMA,
            pltpu.SemaphoreType.DMA,
            pltpu.SemaphoreType.DMA,
            pltpu.SemaphoreType.DMA,
            pltpu.SemaphoreType.DMA,
        ],
    )(_sc_kernel)
    out = run(input_ids.reshape(N).astype(jnp.int32),
              token_type_ids.reshape(N).astype(jnp.int32),
              word_embeddings, position_embeddings, token_type_embeddings,
              gamma, beta)
    return out.reshape(B, L, D)


# speedup vs baseline: 4.2338x; 1.2487x over previous
import functools

import jax
import jax.numpy as jnp
from jax import lax
from jax.experimental import pallas as pl
from jax.experimental.pallas import tpu as pltpu
from jax.experimental.pallas import tpu_sc as plsc

B, L, D, V = 1024, 200, 128, 100000
NC, NS = 2, 16
NW = NC * NS
N = B * L
TPW = N // NW
C = 128
NCHUNK = TPW // C
NPAIR = NCHUNK // 2
NR = D // 16
EPS = 1e-12


def _lane_sum(v):
    iota = lax.broadcasted_iota(jnp.int32, (16,), 0)
    for sh in (8, 4, 2, 1):
        v = v + v.at[iota ^ sh].get(mode="promise_in_bounds")
    return v


def _rsqrt_newton(x):
    i = lax.bitcast_convert_type(x, jnp.int32)
    y = lax.bitcast_convert_type(jnp.int32(0x5F3759DF) - (i >> 1), jnp.float32)
    for _ in range(2):
        y = y * (1.5 - 0.5 * x * y * y)
    return y


def _sc_kernel(ids_hbm, tts_hbm, wemb_hbm, pos_hbm, typ_hbm, gam_hbm, bet_hbm,
               out_hbm,
               idx0, idx1, ttx0, ttx1, x0, x1, y0, y1, pt_v, typ_v, gb_v,
               si0, si1, sg0, sg1, so0, so1):
    idx = (idx0, idx1)
    ttx = (ttx0, ttx1)
    xv = (x0, x1)
    yv = (y0, y1)
    si = (si0, si1)
    sg = (sg0, sg1)
    so = (so0, so1)

    wid = lax.axis_index("s") * NC + lax.axis_index("c")
    wbase = wid * TPW

    pltpu.sync_copy(pos_hbm.at[pl.ds(0, L)], pt_v)
    pltpu.sync_copy(typ_hbm, typ_v)
    pltpu.sync_copy(gam_hbm, gb_v.at[0])
    pltpu.sync_copy(bet_hbm, gb_v.at[1])

    t0 = [typ_v[0, pl.ds(rr * 16, 16)] for rr in range(NR)]
    t1 = [typ_v[1, pl.ds(rr * 16, 16)] for rr in range(NR)]
    dT = [t1[rr] - t0[rr] for rr in range(NR)]
    gv = [gb_v[0, pl.ds(rr * 16, 16)] for rr in range(NR)]
    bv = [gb_v[1, pl.ds(rr * 16, 16)] for rr in range(NR)]

    @pl.loop(0, L)
    def _(i):
        for rr in range(NR):
            s = pl.ds(rr * 16, 16)
            pt_v[i, s] = pt_v[i, s] + t0[rr]

    def launch_inputs(p, c):
        gb = wbase + c * C
        pltpu.async_copy(ids_hbm.at[pl.ds(gb, C)], idx[p], si[p])
        pltpu.async_copy(tts_hbm.at[pl.ds(gb, C)], ttx[p], si[p])

    def wait_inputs(p):
        pltpu.make_async_copy(ids_hbm.at[pl.ds(0, C)], idx[p], si[p]).wait()
        pltpu.make_async_copy(tts_hbm.at[pl.ds(0, C)], ttx[p], si[p]).wait()

    def launch_gather(p):
        pltpu.async_copy(wemb_hbm.at[idx[p]], xv[p], sg[p])

    def wait_gather(p):
        pltpu.make_async_copy(wemb_hbm.at[pl.ds(0, C)], xv[p], sg[p]).wait()

    def launch_out(p, c):
        gb = wbase + c * C
        pltpu.async_copy(yv[p], out_hbm.at[pl.ds(gb, C)], so[p])

    def wait_out(p):
        pltpu.make_async_copy(yv[p], out_hbm.at[pl.ds(0, C)], so[p]).wait()

    def compute(p, c):
        gb = wbase + c * C

        @pl.loop(0, C, unroll=2)
        def _(i):
            pos = lax.rem(gb + i, L)
            g = (i // 16) * 16
            lane = i - g
            ttg = ttx[p][pl.ds(pl.multiple_of(g, 16), 16)]
            tt = ttg.at[jnp.full((16,), lane, jnp.int32)].get(
                mode="promise_in_bounds")
            ttf = tt.astype(jnp.float32)

            xs = []
            s = None
            s2 = None
            for rr in range(NR):
                sl = pl.ds(rr * 16, 16)
                x = (xv[p][i, sl] + pt_v[pos, sl]) + ttf * dT[rr]
                xs.append(x)
                s = x if s is None else s + x
                s2 = x * x if s2 is None else s2 + x * x

            s = _lane_sum(s)
            s2 = _lane_sum(s2)
            mu = s * (1.0 / D)
            var = s2 * (1.0 / D) - mu * mu
            rstd = _rsqrt_newton(var + EPS)

            for rr in range(NR):
                y = ((xs[rr] - mu) * rstd) * gv[rr] + bv[rr]
                yv[p][i, pl.ds(rr * 16, 16)] = y

    def phase(p, c, first=False, no_next=False, no_next2=False):
        if not no_next:
            wait_inputs(1 - p)
            launch_gather(1 - p)
        wait_gather(p)
        if not first:
            wait_out(p)
        compute(p, c)
        launch_out(p, c)
        if not (no_next or no_next2):
            launch_inputs(p, c + 2)

    launch_inputs(0, 0)
    launch_inputs(1, 1)
    wait_inputs(0)
    launch_gather(0)
    phase(0, 0, first=True)
    phase(1, 1, first=True)

    @pl.loop(1, NPAIR - 1)
    def _(k):
        phase(0, 2 * k)
        phase(1, 2 * k + 1)

    phase(0, NCHUNK - 2, no_next2=True)
    phase(1, NCHUNK - 1, no_next=True)
    wait_out(0)
    wait_out(1)


def kernel(input_ids, token_type_ids, word_embeddings, position_embeddings,
           token_type_embeddings, gamma, beta):
    mesh = plsc.VectorSubcoreMesh(core_axis_name="c", subcore_axis_name="s")
    run = functools.partial(
        pl.kernel,
        out_type=jax.ShapeDtypeStruct((N, D), jnp.float32),
        mesh=mesh,
        scratch_types=[
            pltpu.VMEM((C,), jnp.int32),
            pltpu.VMEM((C,), jnp.int32),
            pltpu.VMEM((C,), jnp.int32),
            pltpu.VMEM((C,), jnp.int32),
            pltpu.VMEM((C, D), jnp.float32),
            pltpu.VMEM((C, D), jnp.float32),
            pltpu.VMEM((C, D), jnp.float32),
            pltpu.VMEM((C, D), jnp.float32),
            pltpu.VMEM((L, D), jnp.float32),
            pltpu.VMEM((2, D), jnp.float32),
            pltpu.VMEM((2, D), jnp.float32),
            pltpu.SemaphoreType.DMA,
            pltpu.SemaphoreType.DMA,
            pltpu.SemaphoreType.DMA,
            pltpu.SemaphoreType.DMA,
            pltpu.SemaphoreType.DMA,
            pltpu.SemaphoreType.DMA,
        ],
    )(_sc_kernel)
    out = run(input_ids.reshape(N).astype(jnp.int32),
              token_type_ids.reshape(N).astype(jnp.int32),
              word_embeddings, position_embeddings, token_type_embeddings,
              gamma, beta)
    return out.reshape(B, L, D)


# speedup vs baseline: 6.4566x; 1.5250x over previous
import functools

import jax
import jax.numpy as jnp
from jax import lax
from jax.experimental import pallas as pl
from jax.experimental.pallas import tpu as pltpu
from jax.experimental.pallas import tpu_sc as plsc

B, L, D, V = 1024, 200, 128, 100000
NC, NS = 2, 16
NW = NC * NS
N = B * L
TPW = N // NW
C = 128
NCHUNK = TPW // C
NPAIR = NCHUNK // 2
NR = D // 16
EPS = 1e-12


def _lane_sum(v):
    iota = lax.broadcasted_iota(jnp.int32, (16,), 0)
    for sh in (8, 4, 2, 1):
        v = v + v.at[iota ^ sh].get(mode="promise_in_bounds")
    return v


def _rsqrt_newton(x):
    i = lax.bitcast_convert_type(x, jnp.int32)
    y = lax.bitcast_convert_type(jnp.int32(0x5F3759DF) - (i >> 1), jnp.float32)
    for _ in range(2):
        y = y * (1.5 - 0.5 * x * y * y)
    return y


def _sc_kernel(ids_hbm, tts_hbm, wemb_hbm, pos_hbm, typ_hbm, gam_hbm, bet_hbm,
               out_hbm,
               idx0, idx1, ttx0, ttx1, x0, x1, y0, y1, pt_v, typ_v, gb_v,
               si0, si1, sg0, sg1, so0, so1):
    idx = (idx0, idx1)
    ttx = (ttx0, ttx1)
    xv = (x0, x1)
    yv = (y0, y1)
    si = (si0, si1)
    sg = (sg0, sg1)
    so = (so0, so1)

    wid = lax.axis_index("s") * NC + lax.axis_index("c")
    wbase = wid * TPW

    pltpu.sync_copy(pos_hbm.at[pl.ds(0, L)], pt_v)
    pltpu.sync_copy(typ_hbm, typ_v)
    pltpu.sync_copy(gam_hbm, gb_v.at[0])
    pltpu.sync_copy(bet_hbm, gb_v.at[1])

    t0 = [typ_v[0, pl.ds(rr * 16, 16)] for rr in range(NR)]
    t1 = [typ_v[1, pl.ds(rr * 16, 16)] for rr in range(NR)]
    dT = [t1[rr] - t0[rr] for rr in range(NR)]
    gv = [gb_v[0, pl.ds(rr * 16, 16)] for rr in range(NR)]
    bv = [gb_v[1, pl.ds(rr * 16, 16)] for rr in range(NR)]

    @pl.loop(0, L)
    def _(i):
        for rr in range(NR):
            s = pl.ds(rr * 16, 16)
            pt_v[i, s] = pt_v[i, s] + t0[rr]

    def launch_inputs(p, c):
        gb = wbase + c * C
        pltpu.async_copy(ids_hbm.at[pl.ds(gb, C)], idx[p], si[p])
        pltpu.async_copy(tts_hbm.at[pl.ds(gb, C)], ttx[p], si[p])

    def wait_inputs(p):
        pltpu.make_async_copy(ids_hbm.at[pl.ds(0, C)], idx[p], si[p]).wait()
        pltpu.make_async_copy(tts_hbm.at[pl.ds(0, C)], ttx[p], si[p]).wait()

    def launch_gather(p):
        pltpu.async_copy(wemb_hbm.at[idx[p]], xv[p], sg[p])

    def wait_gather(p):
        pltpu.make_async_copy(wemb_hbm.at[pl.ds(0, C)], xv[p], sg[p]).wait()

    def launch_out(p, c):
        gb = wbase + c * C
        pltpu.async_copy(yv[p], out_hbm.at[pl.ds(gb, C)], so[p])

    def wait_out(p):
        pltpu.make_async_copy(yv[p], out_hbm.at[pl.ds(0, C)], so[p]).wait()

    def compute(p, c):
        gb = wbase + c * C

        def a_stage(i):
            pos = lax.rem(gb + i, L)
            g = (i // 16) * 16
            lane = i - g
            ttg = ttx[p][pl.ds(pl.multiple_of(g, 16), 16)]
            tt = ttg.at[jnp.full((16,), lane, jnp.int32)].get(
                mode="promise_in_bounds")
            ttf = tt.astype(jnp.float32)

            xs = []
            s = None
            s2 = None
            for rr in range(NR):
                sl = pl.ds(rr * 16, 16)
                x = (xv[p][i, sl] + pt_v[pos, sl]) + ttf * dT[rr]
                xs.append(x)
                s = x if s is None else s + x
                s2 = x * x if s2 is None else s2 + x * x
            return (*xs, _lane_sum(s), _lane_sum(s2))

        def b_stage(i, st):
            xs, s, s2 = st[:NR], st[NR], st[NR + 1]
            mu = s * (1.0 / D)
            var = s2 * (1.0 / D) - mu * mu
            rstd = _rsqrt_newton(var + EPS)
            for rr in range(NR):
                y = ((xs[rr] - mu) * rstd) * gv[rr] + bv[rr]
                yv[p][i, pl.ds(rr * 16, 16)] = y

        def body(i, st):
            new = a_stage(i)
            b_stage(i - 1, st)
            return new

        st = lax.fori_loop(1, C, body, a_stage(0))
        b_stage(C - 1, st)

    def phase(p, c, first=False, no_next=False, no_next2=False):
        if not no_next:
            wait_inputs(1 - p)
            launch_gather(1 - p)
        wait_gather(p)
        if not first:
            wait_out(p)
        compute(p, c)
        launch_out(p, c)
        if not (no_next or no_next2):
            launch_inputs(p, c + 2)

    launch_inputs(0, 0)
    launch_inputs(1, 1)
    wait_inputs(0)
    launch_gather(0)
    phase(0, 0, first=True)
    phase(1, 1, first=True)

    @pl.loop(1, NPAIR - 1)
    def _(k):
        phase(0, 2 * k)
        phase(1, 2 * k + 1)

    phase(0, NCHUNK - 2, no_next2=True)
    phase(1, NCHUNK - 1, no_next=True)
    wait_out(0)
    wait_out(1)


def kernel(input_ids, token_type_ids, word_embeddings, position_embeddings,
           token_type_embeddings, gamma, beta):
    mesh = plsc.VectorSubcoreMesh(core_axis_name="c", subcore_axis_name="s")
    run = functools.partial(
        pl.kernel,
        out_type=jax.ShapeDtypeStruct((N, D), jnp.float32),
        mesh=mesh,
        scratch_types=[
            pltpu.VMEM((C,), jnp.int32),
            pltpu.VMEM((C,), jnp.int32),
            pltpu.VMEM((C,), jnp.int32),
            pltpu.VMEM((C,), jnp.int32),
            pltpu.VMEM((C, D), jnp.float32),
            pltpu.VMEM((C, D), jnp.float32),
            pltpu.VMEM((C, D), jnp.float32),
            pltpu.VMEM((C, D), jnp.float32),
            pltpu.VMEM((L, D), jnp.float32),
            pltpu.VMEM((2, D), jnp.float32),
            pltpu.VMEM((2, D), jnp.float32),
            pltpu.SemaphoreType.DMA,
            pltpu.SemaphoreType.DMA,
            pltpu.SemaphoreType.DMA,
            pltpu.SemaphoreType.DMA,
            pltpu.SemaphoreType.DMA,
            pltpu.SemaphoreType.DMA,
        ],
    )(_sc_kernel)
    out = run(input_ids.reshape(N).astype(jnp.int32),
              token_type_ids.reshape(N).astype(jnp.int32),
              word_embeddings, position_embeddings, token_type_embeddings,
              gamma, beta)
    return out.reshape(B, L, D)
